# baseline (speedup 1.0000x reference)
import jax
import jax.numpy as jnp
from jax import lax
from jax.experimental import pallas as pl
from jax.experimental.pallas import tpu as pltpu

_N = 8192
_M = 8192
_ROWS = 64
_NCH = _N // _ROWS
_RING = 8

_CONTRACT = (((1,), (1,)), ((), ()))


def _fused_kernel(x_ref, bias_ref, w_hbm, l_hbm, out_ref,
                  b0, b1, b2, b3, b4, b5, b6, b7, s0, s1, s2, s3, s4, s5, s6, s7, y1_ref):
    bufs = [b0, b1, b2, b3, b4, b5, b6, b7]
    sems = [s0, s1, s2, s3, s4, s5, s6, s7]
    total = 2 * _NCH

    def chunk_ref(j):
        if j < _NCH:
            return w_hbm.at[pl.ds(j * _ROWS, _ROWS)]
        return l_hbm.at[pl.ds((j - _NCH) * _ROWS, _ROWS)]

    copies = [None] * _RING
    for j in range(_RING):
        c = pltpu.make_async_copy(chunk_ref(j), bufs[j], sems[j])
        c.start()
        copies[j] = c

    for i in range(total):
        r = i % _RING
        copies[r].wait()
        if i < _NCH:
            y = lax.dot_general(x_ref[...], bufs[r][...], _CONTRACT,
                                preferred_element_type=jnp.float32)
            y1_ref[:, i * _ROWS:(i + 1) * _ROWS] = y
        else:
            o = lax.dot_general(y1_ref[...], bufs[r][...], _CONTRACT,
                                preferred_element_type=jnp.float32)
            sl = slice((i - _NCH) * _ROWS, (i - _NCH + 1) * _ROWS)
            out_ref[:, sl] = bias_ref[:, sl] + o
        if i + _RING < total:
            c = pltpu.make_async_copy(chunk_ref(i + _RING), bufs[r], sems[r])
            c.start()
            copies[r] = c


def kernel(input, data_lengths, weight, lin_weight, lin_bias):
    x = input.astype(jnp.float32).reshape(1, _M)
    bias = lin_bias.reshape(1, _M).astype(jnp.float32)

    out = pl.pallas_call(
        _fused_kernel,
        in_specs=[
            pl.BlockSpec(memory_space=pltpu.MemorySpace.VMEM),
            pl.BlockSpec(memory_space=pltpu.MemorySpace.VMEM),
            pl.BlockSpec(memory_space=pl.ANY),
            pl.BlockSpec(memory_space=pl.ANY),
        ],
        out_specs=pl.BlockSpec(memory_space=pltpu.MemorySpace.VMEM),
        out_shape=jax.ShapeDtypeStruct((1, _M), jnp.float32),
        scratch_shapes=[pltpu.VMEM((_ROWS, _M), jnp.float32) for _ in range(_RING)]
        + [pltpu.SemaphoreType.DMA for _ in range(_RING)]
        + [pltpu.VMEM((1, _N), jnp.float32)],
    )(x, bias, weight, lin_weight)

    return out.reshape(_M, 1), data_lengths


# speedup vs baseline: 1.1432x; 1.1432x over previous
import jax
import jax.numpy as jnp
from jax import lax
from jax.experimental import pallas as pl
from jax.experimental.pallas import tpu as pltpu

_N = 8192
_M = 8192
_ROWS = 128
_NCH = _N // _ROWS
_RING = 8

_CONTRACT = (((1,), (1,)), ((), ()))


def _fused_kernel(x_ref, bias_ref, w_hbm, l_hbm, out_ref,
                  b0, b1, b2, b3, b4, b5, b6, b7, s0, s1, s2, s3, s4, s5, s6, s7, y1_ref):
    bufs = [b0, b1, b2, b3, b4, b5, b6, b7]
    sems = [s0, s1, s2, s3, s4, s5, s6, s7]
    total = 2 * _NCH

    def chunk_ref(j):
        if j < _NCH:
            return w_hbm.at[pl.ds(j * _ROWS, _ROWS)]
        return l_hbm.at[pl.ds((j - _NCH) * _ROWS, _ROWS)]

    copies = [None] * _RING
    for j in range(_RING):
        c = pltpu.make_async_copy(chunk_ref(j), bufs[j], sems[j])
        c.start()
        copies[j] = c

    for i in range(total):
        r = i % _RING
        copies[r].wait()
        if i < _NCH:
            y = lax.dot_general(x_ref[...], bufs[r][...], _CONTRACT,
                                preferred_element_type=jnp.float32)
            y1_ref[:, i * _ROWS:(i + 1) * _ROWS] = y
        else:
            o = lax.dot_general(y1_ref[...], bufs[r][...], _CONTRACT,
                                preferred_element_type=jnp.float32)
            sl = slice((i - _NCH) * _ROWS, (i - _NCH + 1) * _ROWS)
            out_ref[:, sl] = bias_ref[:, sl] + o
        if i + _RING < total:
            c = pltpu.make_async_copy(chunk_ref(i + _RING), bufs[r], sems[r])
            c.start()
            copies[r] = c


def kernel(input, data_lengths, weight, lin_weight, lin_bias):
    x = input.astype(jnp.float32).reshape(1, _M)
    bias = lin_bias.reshape(1, _M).astype(jnp.float32)

    out = pl.pallas_call(
        _fused_kernel,
        in_specs=[
            pl.BlockSpec(memory_space=pltpu.MemorySpace.VMEM),
            pl.BlockSpec(memory_space=pltpu.MemorySpace.VMEM),
            pl.BlockSpec(memory_space=pl.ANY),
            pl.BlockSpec(memory_space=pl.ANY),
        ],
        out_specs=pl.BlockSpec(memory_space=pltpu.MemorySpace.VMEM),
        out_shape=jax.ShapeDtypeStruct((1, _M), jnp.float32),
        scratch_shapes=[pltpu.VMEM((_ROWS, _M), jnp.float32) for _ in range(_RING)]
        + [pltpu.SemaphoreType.DMA for _ in range(_RING)]
        + [pltpu.VMEM((1, _N), jnp.float32)],
    )(x, bias, weight, lin_weight)

    return out.reshape(_M, 1), data_lengths
